# Initial kernel scaffold; baseline (speedup 1.0000x reference)
import jax
import jax.numpy as jnp
from jax.experimental import pallas as pl


def kernel(x, edge_index, edge_attr, batch, W1, b1, W2, b2, Wc1, bc1, Wc2, bc2):
    raise NotImplementedError("write your pallas kernel here")


# speedup vs baseline: 10.9349x; 10.9349x over previous
import functools

import jax
import jax.numpy as jnp
from jax import lax
from jax.experimental import pallas as pl
from jax.experimental.pallas import tpu as pltpu
from jax.experimental.pallas import tpu_sc as plsc

N = 10000
NP = 10240
E = 320000
D = 128
HD = D // 2
NG = 64
NCLS = 10
NC = 2
NS = 16
NT = NC * NS
CH = 80
EPT = E // NS
NCH = EPT // CH
DEPT = E // NT
DNCH = DEPT // CH
STRIPE = NP // NS


def _mesh():
    return plsc.VectorSubcoreMesh(core_axis_name="c", subcore_axis_name="s",
                                  num_cores=NC, num_subcores=NS)


_SC_PARAMS = pltpu.CompilerParams(use_tc_tiling_on_sc=False,
                                  needs_layout_passes=False)


def _deg_body(dst_hbm, ew_hbm, degp_hbm, dst_v, ew_v, dacc):
    c = lax.axis_index("c")
    s = lax.axis_index("s")
    g = c * NS + s
    pltpu.sync_copy(dst_hbm.at[g], dst_v)
    pltpu.sync_copy(ew_hbm.at[g], ew_v)

    def zero(i, carry):
        dacc[pl.ds(i * 16, 16)] = jnp.zeros((16,), jnp.float32)
        return carry

    lax.fori_loop(0, NP // 16, zero, 0)

    def edge(j, carry):
        for gi in range(CH // 16):
            d16 = dst_v[j, pl.ds(gi * 16, 16)]
            w16 = ew_v[j, pl.ds(gi * 16, 16)]
            plsc.addupdate_scatter(dacc, [d16], w16)
        return carry

    lax.fori_loop(0, DNCH, edge, 0)
    pltpu.sync_copy(dacc, degp_hbm.at[g])


@functools.lru_cache(maxsize=None)
def _deg_call():
    return pl.kernel(
        _deg_body,
        out_type=jax.ShapeDtypeStruct((NT, NP), jnp.float32),
        mesh=_mesh(),
        scratch_types=[
            pltpu.VMEM((DNCH, CH), jnp.int32),
            pltpu.VMEM((DNCH, CH), jnp.float32),
            pltpu.VMEM((NP,), jnp.float32),
        ],
        compiler_params=_SC_PARAMS,
    )


def _spmm_body(xa_hbm, xb_hbm, src_hbm, dst_hbm, ew_hbm, out_hbm,
               src_v, dst_v, ew_v, r0, r1, r2, g0, g1, g2, acc_sh):
    c = lax.axis_index("c")
    s = lax.axis_index("s")
    rows = (r0, r1, r2)
    gsem = (g0, g1, g2)
    pltpu.sync_copy(src_hbm.at[s], src_v)
    pltpu.sync_copy(dst_hbm.at[s], dst_v)
    pltpu.sync_copy(ew_hbm.at[s], ew_v)

    def zero(i, carry):
        for k in range(HD // 16):
            r0[i, pl.ds(k * 16, 16)] = jnp.zeros((16,), jnp.float32)
        return carry

    lax.fori_loop(0, CH, zero, 0)
    base = s * STRIPE
    for b in range(STRIPE // CH):
        pltpu.sync_copy(r0, acc_sh.at[pl.ds(base + b * CH, CH)])

    def issue_gather(j, u):
        @pl.when(c == 0)
        def _():
            pltpu.async_copy(xa_hbm.at[src_v.at[j]], rows[u], gsem[u])

        @pl.when(c == 1)
        def _():
            pltpu.async_copy(xb_hbm.at[src_v.at[j]], rows[u], gsem[u])

    def wait_gather(u):
        @pl.when(c == 0)
        def _():
            pltpu.make_async_copy(xa_hbm.at[src_v.at[0]], rows[u],
                                  gsem[u]).wait()

        @pl.when(c == 1)
        def _():
            pltpu.make_async_copy(xb_hbm.at[src_v.at[0]], rows[u],
                                  gsem[u]).wait()

    issue_gather(0, 0)
    issue_gather(1, 1)
    plsc.subcore_barrier()

    def block(jj, carry):
        j0 = 3 * jj
        wait_gather(0)
        issue_gather(j0 + 2, 2)
        _scale_scatter(j0, 0)
        wait_gather(1)
        issue_gather(j0 + 3, 0)
        _scale_scatter(j0 + 1, 1)
        wait_gather(2)

        @pl.when(jj <= NCH // 3 - 2)
        def _():
            issue_gather(j0 + 4, 1)

        _scale_scatter(j0 + 2, 2)
        return carry

    def _scale_scatter(j, u):
        def scale(gi, carry):
            w16 = ew_v[j, pl.ds(gi * 16, 16)]
            for l in range(16):
                w = w16[l]
                e = gi * 16 + l
                for k in range(HD // 16):
                    rows[u][e, pl.ds(k * 16, 16)] = (
                        rows[u][e, pl.ds(k * 16, 16)] * w)
            return carry

        lax.fori_loop(0, CH // 16, scale, 0)
        pltpu.sync_copy(rows[u], acc_sh.at[dst_v.at[j]], add=True)

    lax.fori_loop(0, NCH // 3, block, 0)
    wait_gather(0)
    _scale_scatter(NCH - 1, 0)
    plsc.subcore_barrier()
    for b in range(STRIPE // CH):
        off = base + b * CH
        pltpu.sync_copy(acc_sh.at[pl.ds(off, CH)], out_hbm.at[c, pl.ds(off, CH)])


@functools.lru_cache(maxsize=None)
def _spmm_call():
    return pl.kernel(
        _spmm_body,
        out_type=jax.ShapeDtypeStruct((NC, NP, HD), jnp.float32),
        mesh=_mesh(),
        scratch_types=[
            pltpu.VMEM((NCH, CH), jnp.int32),
            pltpu.VMEM((NCH, CH), jnp.int32),
            pltpu.VMEM((NCH, CH), jnp.float32),
            pltpu.VMEM((CH, HD), jnp.float32),
            pltpu.VMEM((CH, HD), jnp.float32),
            pltpu.VMEM((CH, HD), jnp.float32),
            pltpu.SemaphoreType.DMA,
            pltpu.SemaphoreType.DMA,
            pltpu.SemaphoreType.DMA,
            pltpu.VMEM_SHARED((NP, HD), jnp.float32),
        ],
        compiler_params=_SC_PARAMS,
    )


def _prep_body(x_ref, w1_ref, degp_ref, xa_ref, xb_ref, dis_ref):
    degt = jnp.transpose(degp_ref[...])
    deg = jnp.sum(degt, axis=1, keepdims=True) + 1.0
    dis = jnp.where(deg > 0, lax.rsqrt(jnp.where(deg > 0, deg, 1.0)), 0.0)
    xw = jnp.dot(x_ref[...], w1_ref[...], preferred_element_type=jnp.float32)
    xws = xw * dis
    xa_ref[...] = xws[:, :HD]
    xb_ref[...] = xws[:, HD:]
    dis_ref[...] = dis


def _prep_call(xp, W1, degp):
    return pl.pallas_call(
        _prep_body,
        out_shape=[jax.ShapeDtypeStruct((NP, HD), jnp.float32),
                   jax.ShapeDtypeStruct((NP, HD), jnp.float32),
                   jax.ShapeDtypeStruct((NP, 1), jnp.float32)],
    )(xp, W1, degp)


def _mid_body(s1_ref, xa_ref, xb_ref, dis_ref, b1_ref, w2_ref,
              xa2_ref, xb2_ref):
    dis = dis_ref[...]
    xws1 = jnp.concatenate([xa_ref[...], xb_ref[...]], axis=1)
    s1 = jnp.concatenate([s1_ref[0], s1_ref[1]], axis=1)
    pre = (s1 + xws1) * dis + b1_ref[...]
    h = jnp.maximum(pre, 0.0)
    xw2 = jnp.dot(h, w2_ref[...], preferred_element_type=jnp.float32)
    xws2 = xw2 * dis
    xa2_ref[...] = xws2[:, :HD]
    xb2_ref[...] = xws2[:, HD:]


def _mid_call(S1, xa, xb, dis, b1r, W2):
    return pl.pallas_call(
        _mid_body,
        out_shape=[jax.ShapeDtypeStruct((NP, HD), jnp.float32),
                   jax.ShapeDtypeStruct((NP, HD), jnp.float32)],
    )(S1, xa, xb, dis, b1r, W2)


def _head_body(s2_ref, xa_ref, xb_ref, dis_ref, b2_ref, bt_ref,
               wc1_ref, bc1_ref, wc2_ref, bc2_ref, out_ref):
    dis = dis_ref[...]
    xws2 = jnp.concatenate([xa_ref[...], xb_ref[...]], axis=1)
    s2 = jnp.concatenate([s2_ref[0], s2_ref[1]], axis=1)
    h2 = (s2 + xws2) * dis + b2_ref[...]
    bt = bt_ref[...]
    gid = lax.broadcasted_iota(jnp.int32, (NG, NP), 0)
    onehot = (gid == bt).astype(jnp.float32)
    sums = jnp.dot(onehot, h2, preferred_element_type=jnp.float32)
    counts = jnp.sum(onehot, axis=1, keepdims=True)
    gmean = sums / jnp.maximum(counts, 1.0)
    z = jnp.maximum(
        jnp.dot(gmean, wc1_ref[...], preferred_element_type=jnp.float32)
        + bc1_ref[...], 0.0)
    out_ref[...] = (jnp.dot(z, wc2_ref[...], preferred_element_type=jnp.float32)
                    + bc2_ref[...])


def _head_call(S2, xa, xb, dis, b2r, bt, Wc1, bc1r, Wc2, bc2r):
    return pl.pallas_call(
        _head_body,
        out_shape=jax.ShapeDtypeStruct((NG, NCLS), jnp.float32),
    )(S2, xa, xb, dis, b2r, bt, Wc1, bc1r, Wc2, bc2r)


def kernel(x, edge_index, edge_attr, batch, W1, b1, W2, b2, Wc1, bc1, Wc2, bc2):
    src_s = edge_index[0].astype(jnp.int32).reshape(NS, NCH, CH)
    dst_s = edge_index[1].astype(jnp.int32).reshape(NS, NCH, CH)
    ew_s = edge_attr.reshape(NS, NCH, CH)
    dst_d = edge_index[1].astype(jnp.int32).reshape(NT, DNCH, CH)
    ew_d = edge_attr.reshape(NT, DNCH, CH)
    xp = jnp.pad(x, ((0, NP - N), (0, 0)))
    bt = jnp.pad(batch.astype(jnp.int32), (0, NP - N),
                 constant_values=NG).reshape(1, NP)

    degp = _deg_call()(dst_d, ew_d)
    xa1, xb1, dis = _prep_call(xp, W1, degp)
    S1 = _spmm_call()(xa1, xb1, src_s, dst_s, ew_s)
    xa2, xb2 = _mid_call(S1, xa1, xb1, dis, b1.reshape(1, -1), W2)
    S2 = _spmm_call()(xa2, xb2, src_s, dst_s, ew_s)
    out = _head_call(S2, xa2, xb2, dis, b2.reshape(1, -1), bt,
                     Wc1, bc1.reshape(1, -1), Wc2, bc2.reshape(1, -1))
    return out


# speedup vs baseline: 28.2514x; 2.5836x over previous
import functools

import jax
import jax.numpy as jnp
from jax import lax
from jax.experimental import pallas as pl
from jax.experimental.pallas import tpu as pltpu
from jax.experimental.pallas import tpu_sc as plsc

N = 10000
E = 320000
D = 128
NG = 64
NCLS = 10
NC = 2
NS = 16
NT = NC * NS
EPT = E // NT
CH = 80
NCH = EPT // CH
STRIPE = N // NS


def _mesh():
    return plsc.VectorSubcoreMesh(core_axis_name="c", subcore_axis_name="s",
                                  num_cores=NC, num_subcores=NS)


_SC_PARAMS = pltpu.CompilerParams(use_tc_tiling_on_sc=False,
                                  needs_layout_passes=False)


def _deg_body(dst_hbm, ew_hbm, degp_hbm, dst_v, ew_v, dacc):
    c = lax.axis_index("c")
    s = lax.axis_index("s")
    g = c * NS + s
    pltpu.sync_copy(dst_hbm.at[g], dst_v)
    pltpu.sync_copy(ew_hbm.at[g], ew_v)

    def zero(i, carry):
        dacc[pl.ds(i * 16, 16)] = jnp.zeros((16,), jnp.float32)
        return carry

    lax.fori_loop(0, N // 16, zero, 0)

    def edge(j, carry):
        for gi in range(CH // 16):
            d16 = dst_v[j, pl.ds(gi * 16, 16)]
            w16 = ew_v[j, pl.ds(gi * 16, 16)]
            plsc.addupdate_scatter(dacc, [d16], w16)
        return carry

    lax.fori_loop(0, NCH, edge, 0)
    pltpu.sync_copy(dacc, degp_hbm.at[g])


@functools.lru_cache(maxsize=None)
def _deg_call():
    return pl.kernel(
        _deg_body,
        out_type=jax.ShapeDtypeStruct((NT, N), jnp.float32),
        mesh=_mesh(),
        scratch_types=[
            pltpu.VMEM((NCH, CH), jnp.int32),
            pltpu.VMEM((NCH, CH), jnp.float32),
            pltpu.VMEM((N,), jnp.float32),
        ],
        compiler_params=_SC_PARAMS,
    )


def _spmm_body(xws_hbm, src_hbm, dst_hbm, ew_hbm, out_hbm,
               src_v, dst_v, ew_v, r0, r1, g0, g1, acc_sh):
    c = lax.axis_index("c")
    s = lax.axis_index("s")
    g = c * NS + s
    rows = (r0, r1)
    gsem = (g0, g1)
    pltpu.sync_copy(src_hbm.at[g], src_v)
    pltpu.sync_copy(dst_hbm.at[g], dst_v)
    pltpu.sync_copy(ew_hbm.at[g], ew_v)

    def zero(i, carry):
        for k in range(D // 16):
            r0[i, pl.ds(k * 16, 16)] = jnp.zeros((16,), jnp.float32)
        return carry

    lax.fori_loop(0, CH, zero, 0)
    base = s * STRIPE
    for b in range(7):
        pltpu.sync_copy(r0, acc_sh.at[pl.ds(base + b * CH, CH)])
    pltpu.sync_copy(r0.at[pl.ds(0, STRIPE - 7 * CH)],
                    acc_sh.at[pl.ds(base + 7 * CH, STRIPE - 7 * CH)])

    def issue_gather(j, u):
        pltpu.async_copy(xws_hbm.at[src_v.at[j]], rows[u], gsem[u])

    def wait_gather(u):
        pltpu.make_async_copy(xws_hbm.at[src_v.at[0]], rows[u],
                              gsem[u]).wait()

    def scale_scatter(j, u):
        def scale(gi, carry):
            w16 = ew_v[j, pl.ds(gi * 16, 16)]
            for l in range(16):
                w = w16[l]
                e = gi * 16 + l
                for k in range(D // 16):
                    rows[u][e, pl.ds(k * 16, 16)] = (
                        rows[u][e, pl.ds(k * 16, 16)] * w)
            return carry

        lax.fori_loop(0, CH // 16, scale, 0)
        pltpu.sync_copy(rows[u], acc_sh.at[dst_v.at[j]], add=True)

    issue_gather(0, 0)
    issue_gather(1, 1)
    plsc.subcore_barrier()

    def block(jj, carry):
        j0 = 2 * jj
        wait_gather(0)
        scale_scatter(j0, 0)
        issue_gather(j0 + 2, 0)
        wait_gather(1)
        scale_scatter(j0 + 1, 1)

        @pl.when(jj <= NCH // 2 - 2)
        def _():
            issue_gather(j0 + 3, 1)

        return carry

    lax.fori_loop(0, NCH // 2, block, 0)
    wait_gather(0)
    scale_scatter(NCH - 1, 0)
    plsc.subcore_barrier()
    for b in range(7):
        off = base + b * CH
        pltpu.sync_copy(acc_sh.at[pl.ds(off, CH)], out_hbm.at[c, pl.ds(off, CH)])
    off = base + 7 * CH
    rem = STRIPE - 7 * CH
    pltpu.sync_copy(acc_sh.at[pl.ds(off, rem)], out_hbm.at[c, pl.ds(off, rem)])


@functools.lru_cache(maxsize=None)
def _spmm_call():
    return pl.kernel(
        _spmm_body,
        out_type=jax.ShapeDtypeStruct((NC, N, D), jnp.float32),
        mesh=_mesh(),
        scratch_types=[
            pltpu.VMEM((NCH, CH), jnp.int32),
            pltpu.VMEM((NCH, CH), jnp.int32),
            pltpu.VMEM((NCH, CH), jnp.float32),
            pltpu.VMEM((CH, D), jnp.float32),
            pltpu.VMEM((CH, D), jnp.float32),
            pltpu.SemaphoreType.DMA,
            pltpu.SemaphoreType.DMA,
            pltpu.VMEM_SHARED((N, D), jnp.float32),
        ],
        compiler_params=_SC_PARAMS,
    )


def _prep_body(x_ref, w1_ref, degp_ref, xws_ref, dis_ref):
    degt = jnp.transpose(degp_ref[...])
    deg = jnp.sum(degt, axis=1, keepdims=True) + 1.0
    dis = jnp.where(deg > 0, lax.rsqrt(jnp.where(deg > 0, deg, 1.0)), 0.0)
    xw = jnp.dot(x_ref[...], w1_ref[...], preferred_element_type=jnp.float32)
    xws_ref[...] = xw * dis
    dis_ref[...] = dis


def _prep_call(x, W1, degp):
    return pl.pallas_call(
        _prep_body,
        out_shape=[jax.ShapeDtypeStruct((N, D), jnp.float32),
                   jax.ShapeDtypeStruct((N, 1), jnp.float32)],
    )(x, W1, degp)


def _mid_body(s1_ref, xws1_ref, dis_ref, b1_ref, w2_ref, xws2_ref):
    dis = dis_ref[...]
    pre = (s1_ref[0] + s1_ref[1] + xws1_ref[...]) * dis + b1_ref[...]
    h = jnp.maximum(pre, 0.0)
    xw2 = jnp.dot(h, w2_ref[...], preferred_element_type=jnp.float32)
    xws2_ref[...] = xw2 * dis


def _mid_call(S1, xws1, dis, b1r, W2):
    return pl.pallas_call(
        _mid_body,
        out_shape=jax.ShapeDtypeStruct((N, D), jnp.float32),
    )(S1, xws1, dis, b1r, W2)


def _head_body(s2_ref, xws2_ref, dis_ref, b2_ref, bt_ref,
               wc1_ref, bc1_ref, wc2_ref, bc2_ref, out_ref):
    dis = dis_ref[...]
    h2 = (s2_ref[0] + s2_ref[1] + xws2_ref[...]) * dis + b2_ref[...]
    bt = bt_ref[...]
    gid = lax.broadcasted_iota(jnp.int32, (NG, N), 0)
    onehot = (gid == bt).astype(jnp.float32)
    sums = jnp.dot(onehot, h2, preferred_element_type=jnp.float32)
    counts = jnp.sum(onehot, axis=1, keepdims=True)
    gmean = sums / jnp.maximum(counts, 1.0)
    z = jnp.maximum(
        jnp.dot(gmean, wc1_ref[...], preferred_element_type=jnp.float32)
        + bc1_ref[...], 0.0)
    out_ref[...] = (jnp.dot(z, wc2_ref[...], preferred_element_type=jnp.float32)
                    + bc2_ref[...])


def _head_call(S2, xws2, dis, b2r, bt, Wc1, bc1r, Wc2, bc2r):
    return pl.pallas_call(
        _head_body,
        out_shape=jax.ShapeDtypeStruct((NG, NCLS), jnp.float32),
    )(S2, xws2, dis, b2r, bt, Wc1, bc1r, Wc2, bc2r)


def kernel(x, edge_index, edge_attr, batch, W1, b1, W2, b2, Wc1, bc1, Wc2, bc2):
    src = edge_index[0].astype(jnp.int32).reshape(NT, NCH, CH)
    dst = edge_index[1].astype(jnp.int32).reshape(NT, NCH, CH)
    ew = edge_attr.reshape(NT, NCH, CH)
    bt = batch.astype(jnp.int32).reshape(1, N)

    degp = _deg_call()(dst, ew)
    xws1, dis = _prep_call(x, W1, degp)
    S1 = _spmm_call()(xws1, src, dst, ew)
    xws2 = _mid_call(S1, xws1, dis, b1.reshape(1, -1), W2)
    S2 = _spmm_call()(xws2, src, dst, ew)
    out = _head_call(S2, xws2, dis, b2.reshape(1, -1), bt,
                     Wc1, bc1.reshape(1, -1), Wc2, bc2.reshape(1, -1))
    return out
